# Initial kernel scaffold; baseline (speedup 1.0000x reference)
import jax
import jax.numpy as jnp
from jax.experimental import pallas as pl


def kernel(user_emb, item_emb, adj_row, adj_col, adj_val, ai_row, ai_col, ai_val, aj_row, aj_col, aj_val):
    raise NotImplementedError("write your pallas kernel here")


# speedup vs baseline: 4.0739x; 4.0739x over previous
import functools

import jax
import jax.numpy as jnp
from jax import lax
from jax.experimental import pallas as pl
from jax.experimental.pallas import tpu as pltpu
from jax.experimental.pallas import tpu_sc as plsc

NC = 2
NS = 16
LANES = 16
NW = NC * NS

K_CHUNK = 128
C_ROWS = 16


@functools.lru_cache(maxsize=None)
def _make_spmm(n_pad, rpw, d, mode):
    nextra = {"plain": 0, "combine": 1, "final": 3}[mode]
    mesh = plsc.VectorSubcoreMesh(core_axis_name="c", subcore_axis_name="s")
    scratch = [
        pltpu.VMEM((rpw, d), jnp.float32),
        pltpu.VMEM((K_CHUNK, d), jnp.float32),
        pltpu.VMEM((K_CHUNK,), jnp.int32),
        pltpu.VMEM((K_CHUNK,), jnp.int32),
        pltpu.VMEM((K_CHUNK,), jnp.float32),
        pltpu.VMEM((LANES,), jnp.int32),
        pltpu.VMEM((max(nextra, 1), C_ROWS, d), jnp.float32),
        pltpu.SemaphoreType.DMA,
    ]

    @functools.partial(
        pl.kernel,
        out_type=jax.ShapeDtypeStruct((n_pad, d), jnp.float32),
        mesh=mesh,
        scratch_types=scratch,
        compiler_params=pltpu.CompilerParams(use_tc_tiling_on_sc=False),
    )
    def spmm(bounds_hbm, row_hbm, col_hbm, val_hbm, x_hbm, *rest):
        extras = rest[:nextra]
        out_hbm, acc, rows_v, col_v, row_v, val_v, bnd_v, ep_v, sem = rest[nextra:]
        wid = lax.axis_index("c") * NS + lax.axis_index("s")
        base_row = wid * rpw

        zeros = jnp.zeros((LANES,), jnp.float32)

        def zbody(i, carry):
            for j in range(d // LANES):
                acc[i, pl.ds(j * LANES, LANES)] = zeros
            return carry

        lax.fori_loop(0, rpw, zbody, 0)

        pltpu.sync_copy(bounds_hbm.at[pl.ds(wid * LANES, LANES)], bnd_v)
        bvec = bnd_v[...]
        lo = bvec[0]
        hi = bvec[1]
        lo8 = (lo // 8) * 8
        nch = (hi - lo8 + (K_CHUNK - 1)) // K_CHUNK
        lane_iota = lax.iota(jnp.int32, LANES)

        def chunk(ci, carry):
            cb = pl.multiple_of(lo8 + ci * K_CHUNK, 8)
            pltpu.sync_copy(col_hbm.at[pl.ds(cb, K_CHUNK)], col_v)
            pltpu.sync_copy(row_hbm.at[pl.ds(cb, K_CHUNK)], row_v)
            pltpu.sync_copy(val_hbm.at[pl.ds(cb, K_CHUNK)], val_v)
            pltpu.async_copy(x_hbm.at[col_v], rows_v, sem).wait()

            def group(g, c2):
                gb = pl.multiple_of(g * LANES, LANES)
                rvec = row_v[pl.ds(gb, LANES)]
                vvec = val_v[pl.ds(gb, LANES)]
                eglob = cb + gb + lane_iota
                mask = (eglob >= lo) & (eglob < hi)
                vvec = jnp.where(mask, vvec, 0.0)
                ovec = jnp.clip(rvec - base_row, 0, rpw - 1)
                for l in range(LANES):
                    o = ovec[l]
                    vv = jnp.full((LANES,), vvec[l], jnp.float32)
                    e = g * LANES + l
                    for j in range(d // LANES):
                        xr = rows_v[e, pl.ds(j * LANES, LANES)]
                        plsc.addupdate(acc.at[o, pl.ds(j * LANES, LANES)],
                                       xr * vv)
                return c2

            lax.fori_loop(0, K_CHUNK // LANES, group, 0)
            return carry

        lax.fori_loop(0, nch, chunk, 0)

        if mode == "combine":
            (a_hbm,) = extras

            def ep(t, carry):
                r0 = t * C_ROWS
                pltpu.sync_copy(
                    a_hbm.at[pl.ds(base_row + r0, C_ROWS)], ep_v.at[0]
                )

                def vbody(rr, c2):
                    for j in range(d // LANES):
                        js = pl.ds(j * LANES, LANES)
                        av = ep_v[0, rr, js]
                        acc[r0 + rr, js] = 1.1 * av - 0.1 * acc[r0 + rr, js]
                    return c2

                lax.fori_loop(0, C_ROWS, vbody, 0)
                return carry

            lax.fori_loop(0, rpw // C_ROWS, ep, 0)
        elif mode == "final":
            e0_hbm, e1_hbm, e2_hbm = extras

            def ep(t, carry):
                r0 = t * C_ROWS
                pltpu.sync_copy(e0_hbm.at[pl.ds(base_row + r0, C_ROWS)], ep_v.at[0])
                pltpu.sync_copy(e1_hbm.at[pl.ds(base_row + r0, C_ROWS)], ep_v.at[1])
                pltpu.sync_copy(e2_hbm.at[pl.ds(base_row + r0, C_ROWS)], ep_v.at[2])

                def vbody(rr, c2):
                    for j in range(d // LANES):
                        js = pl.ds(j * LANES, LANES)
                        s = (ep_v[0, rr, js] + ep_v[1, rr, js]
                             + ep_v[2, rr, js] + acc[r0 + rr, js])
                        acc[r0 + rr, js] = 0.25 * s
                    return c2

                lax.fori_loop(0, C_ROWS, vbody, 0)
                return carry

            lax.fori_loop(0, rpw // C_ROWS, ep, 0)

        pltpu.sync_copy(acc, out_hbm.at[pl.ds(base_row, rpw)])

    return spmm


def kernel(user_emb, item_emb, adj_row, adj_col, adj_val,
           ai_row, ai_col, ai_val, aj_row, aj_col, aj_val):
    n = user_emb.shape[0] + item_emb.shape[0]
    user_num = user_emb.shape[0]
    d = user_emb.shape[1]
    rpw = -(-n // NW)
    rpw = -(-rpw // C_ROWS) * C_ROWS
    n_pad = NW * rpw

    def prep(row, col, val):
        row = row.astype(jnp.int32)
        col = col.astype(jnp.int32)
        val = val.astype(jnp.float32)
        cuts = (jnp.arange(NW + 1, dtype=jnp.int32) * rpw).astype(row.dtype)
        b = jnp.searchsorted(row, cuts).astype(jnp.int32)
        bounds = jnp.zeros((NW, LANES), jnp.int32)
        bounds = bounds.at[:, 0].set(b[:-1]).at[:, 1].set(b[1:]).reshape(-1)
        pad = K_CHUNK + 8
        row = jnp.pad(row, (0, pad))
        col = jnp.pad(col, (0, pad))
        val = jnp.pad(val, (0, pad))
        return bounds, row, col, val

    adj = prep(adj_row, adj_col, adj_val)
    aj = prep(aj_row, aj_col, aj_val)
    ai = prep(ai_row, ai_col, ai_val)

    x0 = jnp.concatenate([user_emb, item_emb], axis=0)
    x0 = jnp.pad(x0, ((0, n_pad - n), (0, 0)))

    plain = _make_spmm(n_pad, rpw, d, "plain")
    combine = _make_spmm(n_pad, rpw, d, "combine")
    final = _make_spmm(n_pad, rpw, d, "final")

    a1 = plain(*adj, x0)
    b1 = plain(*aj, a1)
    e1 = combine(*ai, b1, a1)
    a2 = plain(*adj, e1)
    b2 = plain(*aj, a2)
    e2 = combine(*ai, b2, a2)
    out = final(*adj, e2, x0, e1, e2)

    return (out[:user_num], out[user_num:n])


# speedup vs baseline: 6.1857x; 1.5184x over previous
import functools

import jax
import jax.numpy as jnp
from jax import lax
from jax.experimental import pallas as pl
from jax.experimental.pallas import tpu as pltpu
from jax.experimental.pallas import tpu_sc as plsc

NC = 2
NS = 16
LANES = 16
NW = NC * NS

K_CHUNK = 128
C_ROWS = 16


@functools.lru_cache(maxsize=None)
def _make_spmm(n_pad, rpw, d, mode):
    nextra = {"plain": 0, "combine": 1, "final": 3}[mode]
    mesh = plsc.VectorSubcoreMesh(core_axis_name="c", subcore_axis_name="s")
    scratch = [
        pltpu.VMEM_SHARED((NS * rpw, d), jnp.float32),
        pltpu.VMEM((2, K_CHUNK, d), jnp.float32),
        pltpu.VMEM((2, K_CHUNK), jnp.int32),
        pltpu.VMEM((2, K_CHUNK), jnp.int32),
        pltpu.VMEM((2, K_CHUNK), jnp.float32),
        pltpu.VMEM((2, K_CHUNK), jnp.int32),
        pltpu.VMEM((LANES,), jnp.int32),
        pltpu.VMEM((4, C_ROWS, d), jnp.float32),
        pltpu.SemaphoreType.DMA,
        pltpu.SemaphoreType.DMA,
        pltpu.SemaphoreType.DMA,
        pltpu.SemaphoreType.DMA,
        pltpu.SemaphoreType.DMA,
    ]

    @functools.partial(
        pl.kernel,
        out_type=jax.ShapeDtypeStruct((n_pad, d), jnp.float32),
        mesh=mesh,
        scratch_types=scratch,
        compiler_params=pltpu.CompilerParams(use_tc_tiling_on_sc=False),
    )
    def spmm(bounds_hbm, row_hbm, col_hbm, val_hbm, x_hbm, *rest):
        extras = rest[:nextra]
        (out_hbm, acc, rows_v, col_v, row_v, val_v, roff_v, bnd_v, ep_v,
         isem0, isem1, gsem0, gsem1, ssem) = rest[nextra:]
        isem = (isem0, isem1)
        gsem = (gsem0, gsem1)
        sid = lax.axis_index("s")
        wid = lax.axis_index("c") * NS + sid
        base_row = wid * rpw
        sbase = sid * rpw

        zeros = jnp.zeros((LANES,), jnp.float32)

        def zv(i, carry):
            for j in range(d // LANES):
                rows_v[0, i, pl.ds(j * LANES, LANES)] = zeros
            return carry

        lax.fori_loop(0, K_CHUNK, zv, 0)
        nfull = rpw // K_CHUNK
        rem = rpw % K_CHUNK
        for t in range(nfull):
            pltpu.sync_copy(rows_v.at[0],
                            acc.at[pl.ds(sbase + t * K_CHUNK, K_CHUNK)])
        if rem:
            pltpu.sync_copy(rows_v.at[0, pl.ds(0, rem)],
                            acc.at[pl.ds(sbase + nfull * K_CHUNK, rem)])

        pltpu.sync_copy(bounds_hbm.at[pl.ds(wid * LANES, LANES)], bnd_v)
        bvec = bnd_v[...]
        lo = bvec[0]
        hi = bvec[1]
        lo8 = (lo // 8) * 8
        nch = (hi - lo8 + (K_CHUNK - 1)) // K_CHUNK
        lane_iota = lax.iota(jnp.int32, LANES)

        def chunk_base(ci):
            return pl.multiple_of(lo8 + ci * K_CHUNK, 8)

        def start_idx(ci, b):
            cb = chunk_base(ci)
            pltpu.async_copy(col_hbm.at[pl.ds(cb, K_CHUNK)], col_v.at[b], isem[b])
            pltpu.async_copy(row_hbm.at[pl.ds(cb, K_CHUNK)], row_v.at[b], isem[b])
            pltpu.async_copy(val_hbm.at[pl.ds(cb, K_CHUNK)], val_v.at[b], isem[b])

        def wait_idx(b):
            pltpu.make_async_copy(col_hbm.at[pl.ds(0, K_CHUNK)], col_v.at[b],
                                  isem[b]).wait()
            pltpu.make_async_copy(row_hbm.at[pl.ds(0, K_CHUNK)], row_v.at[b],
                                  isem[b]).wait()
            pltpu.make_async_copy(val_hbm.at[pl.ds(0, K_CHUNK)], val_v.at[b],
                                  isem[b]).wait()

        def start_gather(b):
            pltpu.async_copy(x_hbm.at[col_v.at[b]], rows_v.at[b], gsem[b])

        def wait_gather(b):
            pltpu.make_async_copy(x_hbm.at[col_v.at[b]], rows_v.at[b],
                                  gsem[b]).wait()

        def start_scatter(b):
            pltpu.async_copy(rows_v.at[b], acc.at[roff_v.at[b]], ssem, add=True)

        def wait_scatter():
            pltpu.make_async_copy(rows_v.at[0], acc.at[roff_v.at[0]],
                                  ssem).wait()

        def compute(ci, b):
            cb = chunk_base(ci)

            def group(g, c2):
                gb = pl.multiple_of(g * LANES, LANES)
                rvec = row_v[b, pl.ds(gb, LANES)]
                vvec = val_v[b, pl.ds(gb, LANES)]
                eglob = cb + gb + lane_iota
                mask = (eglob >= lo) & (eglob < hi)
                vvec = jnp.where(mask, vvec, 0.0)
                roff_v[b, pl.ds(gb, LANES)] = sbase + jnp.clip(
                    rvec - base_row, 0, rpw - 1)
                for l in range(LANES):
                    vv = jnp.full((LANES,), vvec[l], jnp.float32)
                    e = gb + l
                    for j in range(d // LANES):
                        js = pl.ds(j * LANES, LANES)
                        rows_v[b, e, js] = rows_v[b, e, js] * vv
                return c2

            lax.fori_loop(0, K_CHUNK // LANES, group, 0)

        @pl.when(nch > 0)
        def _prologue():
            start_idx(0, 0)

            @pl.when(nch > 1)
            def _():
                start_idx(1, 1)

            wait_idx(0)
            start_gather(0)

        def pair(p, carry):
            for b in range(2):
                ci = p * 2 + b

                @pl.when(ci < nch)
                def _(ci=ci, b=b):
                    wait_gather(b)
                    compute(ci, b)

                    @pl.when(ci >= 1)
                    def _():
                        wait_scatter()

                    start_scatter(b)

                    @pl.when(ci + 1 < nch)
                    def _(b=b):
                        wait_idx(1 - b)
                        start_gather(1 - b)

                    @pl.when(ci + 2 < nch)
                    def _(ci=ci, b=b):
                        start_idx(ci + 2, b)

            return carry

        lax.fori_loop(0, (nch + 1) // 2, pair, 0)

        @pl.when(nch > 0)
        def _drain():
            wait_scatter()

        nchk = rpw // C_ROWS
        if mode == "plain":
            pltpu.sync_copy(acc.at[pl.ds(sbase, rpw)],
                            out_hbm.at[pl.ds(base_row, rpw)])
        elif mode == "combine":
            (a_hbm,) = extras

            def ep(t, carry):
                r0 = t * C_ROWS
                pltpu.sync_copy(a_hbm.at[pl.ds(base_row + r0, C_ROWS)],
                                ep_v.at[0])
                pltpu.sync_copy(acc.at[pl.ds(sbase + r0, C_ROWS)], ep_v.at[3])

                def vbody(rr, c2):
                    for j in range(d // LANES):
                        js = pl.ds(j * LANES, LANES)
                        ep_v[3, rr, js] = (1.1 * ep_v[0, rr, js]
                                           - 0.1 * ep_v[3, rr, js])
                    return c2

                lax.fori_loop(0, C_ROWS, vbody, 0)
                pltpu.sync_copy(ep_v.at[3],
                                out_hbm.at[pl.ds(base_row + r0, C_ROWS)])
                return carry

            lax.fori_loop(0, nchk, ep, 0)
        else:
            e0_hbm, e1_hbm, e2_hbm = extras

            def ep(t, carry):
                r0 = t * C_ROWS
                pltpu.sync_copy(e0_hbm.at[pl.ds(base_row + r0, C_ROWS)], ep_v.at[0])
                pltpu.sync_copy(e1_hbm.at[pl.ds(base_row + r0, C_ROWS)], ep_v.at[1])
                pltpu.sync_copy(e2_hbm.at[pl.ds(base_row + r0, C_ROWS)], ep_v.at[2])
                pltpu.sync_copy(acc.at[pl.ds(sbase + r0, C_ROWS)], ep_v.at[3])

                def vbody(rr, c2):
                    for j in range(d // LANES):
                        js = pl.ds(j * LANES, LANES)
                        s = (ep_v[0, rr, js] + ep_v[1, rr, js]
                             + ep_v[2, rr, js] + ep_v[3, rr, js])
                        ep_v[3, rr, js] = 0.25 * s
                    return c2

                lax.fori_loop(0, C_ROWS, vbody, 0)
                pltpu.sync_copy(ep_v.at[3],
                                out_hbm.at[pl.ds(base_row + r0, C_ROWS)])
                return carry

            lax.fori_loop(0, nchk, ep, 0)

    return spmm


def kernel(user_emb, item_emb, adj_row, adj_col, adj_val,
           ai_row, ai_col, ai_val, aj_row, aj_col, aj_val):
    n = user_emb.shape[0] + item_emb.shape[0]
    user_num = user_emb.shape[0]
    d = user_emb.shape[1]
    rpw = -(-n // NW)
    rpw = -(-rpw // C_ROWS) * C_ROWS
    n_pad = NW * rpw

    def prep(row, col, val):
        row = row.astype(jnp.int32)
        col = col.astype(jnp.int32)
        val = val.astype(jnp.float32)
        cuts = (jnp.arange(NW + 1, dtype=jnp.int32) * rpw).astype(row.dtype)
        b = jnp.searchsorted(row, cuts).astype(jnp.int32)
        bounds = jnp.zeros((NW, LANES), jnp.int32)
        bounds = bounds.at[:, 0].set(b[:-1]).at[:, 1].set(b[1:]).reshape(-1)
        pad = 2 * K_CHUNK + 8
        row = jnp.pad(row, (0, pad))
        col = jnp.pad(col, (0, pad))
        val = jnp.pad(val, (0, pad))
        return bounds, row, col, val

    adj = prep(adj_row, adj_col, adj_val)
    aj = prep(aj_row, aj_col, aj_val)
    ai = prep(ai_row, ai_col, ai_val)

    x0 = jnp.concatenate([user_emb, item_emb], axis=0)
    x0 = jnp.pad(x0, ((0, n_pad - n), (0, 0)))

    plain = _make_spmm(n_pad, rpw, d, "plain")
    combine = _make_spmm(n_pad, rpw, d, "combine")
    final = _make_spmm(n_pad, rpw, d, "final")

    a1 = plain(*adj, x0)
    b1 = plain(*aj, a1)
    e1 = combine(*ai, b1, a1)
    a2 = plain(*adj, e1)
    b2 = plain(*aj, a2)
    e2 = combine(*ai, b2, a2)
    out = final(*adj, e2, x0, e1, e2)

    return (out[:user_num], out[user_num:n])


# speedup vs baseline: 11.6848x; 1.8890x over previous
import functools

import jax
import jax.numpy as jnp
from jax import lax
from jax.experimental import pallas as pl
from jax.experimental.pallas import tpu as pltpu
from jax.experimental.pallas import tpu_sc as plsc

NC = 2
NS = 16
LANES = 16
NW = NC * NS
NPASS = 2

K_CHUNK = 128
C_ROWS = 16


@functools.lru_cache(maxsize=None)
def _make_spmm(n_pad, rpw, d, mode):
    nextra = {"plain": 0, "combine": 1, "final": 3}[mode]
    mesh = plsc.VectorSubcoreMesh(core_axis_name="c", subcore_axis_name="s")
    scratch = [
        pltpu.VMEM_SHARED((NS * rpw, d), jnp.float32),
        pltpu.VMEM((2, K_CHUNK, d), jnp.float32),
        pltpu.VMEM((2, K_CHUNK, d), jnp.float32),
        pltpu.VMEM((2, K_CHUNK), jnp.int32),
        pltpu.VMEM((2, K_CHUNK), jnp.int32),
        pltpu.VMEM((2, K_CHUNK), jnp.float32),
        pltpu.VMEM((2, K_CHUNK), jnp.int32),
        pltpu.VMEM((LANES,), jnp.int32),
        pltpu.VMEM((4, C_ROWS, d), jnp.float32),
        pltpu.SemaphoreType.DMA,
        pltpu.SemaphoreType.DMA,
        pltpu.SemaphoreType.DMA,
        pltpu.SemaphoreType.DMA,
        pltpu.SemaphoreType.DMA,
    ]

    @functools.partial(
        pl.kernel,
        out_type=jax.ShapeDtypeStruct((n_pad, d), jnp.float32),
        mesh=mesh,
        scratch_types=scratch,
        compiler_params=pltpu.CompilerParams(use_tc_tiling_on_sc=False),
    )
    def spmm(bounds_hbm, row_hbm, col_hbm, val_hbm, x_hbm, *rest):
        extras = rest[:nextra]
        (out_hbm, acc, rows_v, scaled_v, col_v, row_v, val_v, roff_v, bnd_v,
         ep_v, isem0, isem1, gsem0, gsem1, ssem) = rest[nextra:]
        isem = (isem0, isem1)
        gsem = (gsem0, gsem1)
        sid = lax.axis_index("s")
        wid = lax.axis_index("c") * NS + sid
        sbase = sid * rpw

        zeros = jnp.zeros((LANES,), jnp.float32)
        lane_iota = lax.iota(jnp.int32, LANES)
        nfull = rpw // K_CHUNK
        rem = rpw % K_CHUNK

        def pass_body(p, pcarry):
            vwid = p * NW + wid
            base_row = vwid * rpw

            def zv(i, carry):
                for j in range(d // LANES):
                    scaled_v[0, i, pl.ds(j * LANES, LANES)] = zeros
                return carry

            lax.fori_loop(0, K_CHUNK, zv, 0)
            for t in range(nfull):
                pltpu.sync_copy(scaled_v.at[0],
                                acc.at[pl.ds(sbase + t * K_CHUNK, K_CHUNK)])
            if rem:
                pltpu.sync_copy(scaled_v.at[0, pl.ds(0, rem)],
                                acc.at[pl.ds(sbase + nfull * K_CHUNK, rem)])

            pltpu.sync_copy(bounds_hbm.at[pl.ds(vwid * LANES, LANES)], bnd_v)
            bvec = bnd_v[...]
            lo = bvec[0]
            hi = bvec[1]
            lo8 = (lo // 8) * 8
            nch = (hi - lo8 + (K_CHUNK - 1)) // K_CHUNK

            def chunk_base(ci):
                return pl.multiple_of(lo8 + ci * K_CHUNK, 8)

            def start_idx(ci, b):
                cb = chunk_base(ci)
                pltpu.async_copy(col_hbm.at[pl.ds(cb, K_CHUNK)], col_v.at[b],
                                 isem[b])
                pltpu.async_copy(row_hbm.at[pl.ds(cb, K_CHUNK)], row_v.at[b],
                                 isem[b])
                pltpu.async_copy(val_hbm.at[pl.ds(cb, K_CHUNK)], val_v.at[b],
                                 isem[b])

            def wait_idx(b):
                pltpu.make_async_copy(col_hbm.at[pl.ds(0, K_CHUNK)],
                                      col_v.at[b], isem[b]).wait()
                pltpu.make_async_copy(row_hbm.at[pl.ds(0, K_CHUNK)],
                                      row_v.at[b], isem[b]).wait()
                pltpu.make_async_copy(val_hbm.at[pl.ds(0, K_CHUNK)],
                                      val_v.at[b], isem[b]).wait()

            def start_gather(b):
                pltpu.async_copy(x_hbm.at[col_v.at[b]], rows_v.at[b], gsem[b])

            def wait_gather(b):
                pltpu.make_async_copy(x_hbm.at[col_v.at[b]], rows_v.at[b],
                                      gsem[b]).wait()

            def start_scatter(b):
                pltpu.async_copy(scaled_v.at[b], acc.at[roff_v.at[b]], ssem,
                                 add=True)

            def wait_scatter():
                pltpu.make_async_copy(scaled_v.at[0], acc.at[roff_v.at[0]],
                                      ssem).wait()

            def compute(ci, b):
                cb = chunk_base(ci)

                @plsc.parallel_loop(0, K_CHUNK // LANES, 1, unroll=2)
                def group(g):
                    gb = pl.multiple_of(g * LANES, LANES)
                    rvec = row_v[b, pl.ds(gb, LANES)]
                    vvec = val_v[b, pl.ds(gb, LANES)]
                    eglob = cb + gb + lane_iota
                    mask = (eglob >= lo) & (eglob < hi)
                    vvec = jnp.where(mask, vvec, 0.0)
                    roff_v[b, pl.ds(gb, LANES)] = sbase + jnp.clip(
                        rvec - base_row, 0, rpw - 1)
                    for l in range(LANES):
                        vv = jnp.full((LANES,), vvec[l], jnp.float32)
                        e = gb + l
                        for j in range(d // LANES):
                            js = pl.ds(j * LANES, LANES)
                            scaled_v[b, e, js] = rows_v[b, e, js] * vv

            @pl.when(nch > 0)
            def _prologue():
                start_idx(0, 0)

                @pl.when(nch > 1)
                def _():
                    start_idx(1, 1)

                wait_idx(0)
                start_gather(0)

            def pair(pp, carry):
                for b in range(2):
                    ci = pp * 2 + b

                    @pl.when(ci < nch)
                    def _(ci=ci, b=b):
                        wait_gather(b)
                        compute(ci, b)

                        @pl.when(ci >= 1)
                        def _():
                            wait_scatter()

                        start_scatter(b)

                        @pl.when(ci + 1 < nch)
                        def _(b=b):
                            wait_idx(1 - b)
                            start_gather(1 - b)

                        @pl.when(ci + 2 < nch)
                        def _(ci=ci, b=b):
                            start_idx(ci + 2, b)

                return carry

            lax.fori_loop(0, (nch + 1) // 2, pair, 0)

            @pl.when(nch > 0)
            def _drain():
                wait_scatter()

            nchk = rpw // C_ROWS
            if mode == "plain":
                pltpu.sync_copy(acc.at[pl.ds(sbase, rpw)],
                                out_hbm.at[pl.ds(base_row, rpw)])
            elif mode == "combine":
                (a_hbm,) = extras

                def ep(t, carry):
                    r0 = t * C_ROWS
                    pltpu.sync_copy(a_hbm.at[pl.ds(base_row + r0, C_ROWS)],
                                    ep_v.at[0])
                    pltpu.sync_copy(acc.at[pl.ds(sbase + r0, C_ROWS)],
                                    ep_v.at[3])

                    def vbody(rr, c2):
                        for j in range(d // LANES):
                            js = pl.ds(j * LANES, LANES)
                            ep_v[3, rr, js] = (1.1 * ep_v[0, rr, js]
                                               - 0.1 * ep_v[3, rr, js])
                        return c2

                    lax.fori_loop(0, C_ROWS, vbody, 0)
                    pltpu.sync_copy(ep_v.at[3],
                                    out_hbm.at[pl.ds(base_row + r0, C_ROWS)])
                    return carry

                lax.fori_loop(0, nchk, ep, 0)
            else:
                e0_hbm, e1_hbm, e2_hbm = extras

                def ep(t, carry):
                    r0 = t * C_ROWS
                    gr = base_row + r0
                    pltpu.sync_copy(e0_hbm.at[pl.ds(gr, C_ROWS)], ep_v.at[0])
                    pltpu.sync_copy(e1_hbm.at[pl.ds(gr, C_ROWS)], ep_v.at[1])
                    pltpu.sync_copy(e2_hbm.at[pl.ds(gr, C_ROWS)], ep_v.at[2])
                    pltpu.sync_copy(acc.at[pl.ds(sbase + r0, C_ROWS)],
                                    ep_v.at[3])

                    def vbody(rr, c2):
                        for j in range(d // LANES):
                            js = pl.ds(j * LANES, LANES)
                            s = (ep_v[0, rr, js] + ep_v[1, rr, js]
                                 + ep_v[2, rr, js] + ep_v[3, rr, js])
                            ep_v[3, rr, js] = 0.25 * s
                        return c2

                    lax.fori_loop(0, C_ROWS, vbody, 0)
                    pltpu.sync_copy(ep_v.at[3],
                                    out_hbm.at[pl.ds(gr, C_ROWS)])
                    return carry

                lax.fori_loop(0, nchk, ep, 0)

            return pcarry

        lax.fori_loop(0, NPASS, pass_body, 0)

    return spmm


def kernel(user_emb, item_emb, adj_row, adj_col, adj_val,
           ai_row, ai_col, ai_val, aj_row, aj_col, aj_val):
    n = user_emb.shape[0] + item_emb.shape[0]
    user_num = user_emb.shape[0]
    d = user_emb.shape[1]
    nvw = NW * NPASS
    rpw = -(-n // nvw)
    rpw = -(-rpw // C_ROWS) * C_ROWS
    n_pad = nvw * rpw

    def prep(row, col, val):
        row = row.astype(jnp.int32)
        col = col.astype(jnp.int32)
        val = val.astype(jnp.float32)
        cuts = (jnp.arange(nvw + 1, dtype=jnp.int32) * rpw).astype(row.dtype)
        b = jnp.searchsorted(row, cuts).astype(jnp.int32)
        bounds = jnp.zeros((nvw, LANES), jnp.int32)
        bounds = bounds.at[:, 0].set(b[:-1]).at[:, 1].set(b[1:]).reshape(-1)
        pad = 2 * K_CHUNK + 8
        row = jnp.pad(row, (0, pad))
        col = jnp.pad(col, (0, pad))
        val = jnp.pad(val, (0, pad))
        return bounds, row, col, val

    adj = prep(adj_row, adj_col, adj_val)
    aj = prep(aj_row, aj_col, aj_val)
    ai = prep(ai_row, ai_col, ai_val)

    x0 = jnp.concatenate([user_emb, item_emb], axis=0)
    x0 = jnp.pad(x0, ((0, n_pad - n), (0, 0)))

    plain = _make_spmm(n_pad, rpw, d, "plain")
    combine = _make_spmm(n_pad, rpw, d, "combine")
    final = _make_spmm(n_pad, rpw, d, "final")

    a1 = plain(*adj, x0)
    b1 = plain(*aj, a1)
    e1 = combine(*ai, b1, a1)
    a2 = plain(*adj, e1)
    b2 = plain(*aj, a2)
    e2 = combine(*ai, b2, a2)
    out = final(*adj, e2, x0, e1, e2)

    return (out[:user_num], out[user_num:n])


# speedup vs baseline: 17.1113x; 1.4644x over previous
import functools

import jax
import jax.numpy as jnp
from jax import lax
from jax.experimental import pallas as pl
from jax.experimental.pallas import tpu as pltpu
from jax.experimental.pallas import tpu_sc as plsc

NC = 2
NS = 16
LANES = 16
NW = NC * NS
NPASS = 2
NBUF = 4

K_CHUNK = 128
EP_CHUNKS = 8


@functools.lru_cache(maxsize=None)
def _make_spmm(n_pad, rpw, d, mode):
    nextra = {"plain": 0, "combine": 1, "final": 3}[mode]
    ep_rows = rpw // EP_CHUNKS
    assert rpw % EP_CHUNKS == 0
    mesh = plsc.VectorSubcoreMesh(core_axis_name="c", subcore_axis_name="s")
    scratch = [
        pltpu.VMEM_SHARED((NS * rpw, d), jnp.float32),
        pltpu.VMEM((NBUF, K_CHUNK, d), jnp.float32),
        pltpu.VMEM((2, K_CHUNK, d), jnp.float32),
        pltpu.VMEM((NBUF, K_CHUNK), jnp.int32),
        pltpu.VMEM((NBUF, K_CHUNK), jnp.int32),
        pltpu.VMEM((NBUF, K_CHUNK), jnp.float32),
        pltpu.VMEM((2, K_CHUNK), jnp.int32),
        pltpu.VMEM((LANES,), jnp.int32),
        pltpu.VMEM((4, ep_rows, d), jnp.float32),
        [pltpu.SemaphoreType.DMA] * NBUF,
        [pltpu.SemaphoreType.DMA] * NBUF,
        pltpu.SemaphoreType.DMA,
    ]

    @functools.partial(
        pl.kernel,
        out_type=jax.ShapeDtypeStruct((n_pad, d), jnp.float32),
        mesh=mesh,
        scratch_types=scratch,
        compiler_params=pltpu.CompilerParams(use_tc_tiling_on_sc=False),
    )
    def spmm(bounds_hbm, row_hbm, col_hbm, val_hbm, x_hbm, *rest):
        extras = rest[:nextra]
        (out_hbm, acc, rows_v, scaled_v, col_v, row_v, val_v, roff_v, bnd_v,
         ep_v, isem, gsem, ssem) = rest[nextra:]
        sid = lax.axis_index("s")
        wid = lax.axis_index("c") * NS + sid
        sbase = sid * rpw

        zeros = jnp.zeros((LANES,), jnp.float32)
        lane_iota = lax.iota(jnp.int32, LANES)
        nfull = rpw // K_CHUNK
        rem = rpw % K_CHUNK

        def pass_body(p, pcarry):
            vwid = p * NW + wid
            base_row = vwid * rpw

            def zv(i, carry):
                for j in range(d // LANES):
                    scaled_v[0, i, pl.ds(j * LANES, LANES)] = zeros
                return carry

            lax.fori_loop(0, K_CHUNK, zv, 0)
            for t in range(nfull):
                pltpu.sync_copy(scaled_v.at[0],
                                acc.at[pl.ds(sbase + t * K_CHUNK, K_CHUNK)])
            if rem:
                pltpu.sync_copy(scaled_v.at[0, pl.ds(0, rem)],
                                acc.at[pl.ds(sbase + nfull * K_CHUNK, rem)])

            pltpu.sync_copy(bounds_hbm.at[pl.ds(vwid * LANES, LANES)], bnd_v)
            bvec = bnd_v[...]
            lo = bvec[0]
            hi = bvec[1]
            lo8 = (lo // 8) * 8
            nch = (hi - lo8 + (K_CHUNK - 1)) // K_CHUNK

            def chunk_base(ci):
                return pl.multiple_of(lo8 + ci * K_CHUNK, 8)

            def start_idx(ci, b):
                cb = chunk_base(ci)
                pltpu.async_copy(col_hbm.at[pl.ds(cb, K_CHUNK)], col_v.at[b],
                                 isem[b])
                pltpu.async_copy(row_hbm.at[pl.ds(cb, K_CHUNK)], row_v.at[b],
                                 isem[b])
                pltpu.async_copy(val_hbm.at[pl.ds(cb, K_CHUNK)], val_v.at[b],
                                 isem[b])

            def wait_idx(b):
                pltpu.make_async_copy(col_hbm.at[pl.ds(0, K_CHUNK)],
                                      col_v.at[b], isem[b]).wait()
                pltpu.make_async_copy(row_hbm.at[pl.ds(0, K_CHUNK)],
                                      row_v.at[b], isem[b]).wait()
                pltpu.make_async_copy(val_hbm.at[pl.ds(0, K_CHUNK)],
                                      val_v.at[b], isem[b]).wait()

            def start_gather(b):
                pltpu.async_copy(x_hbm.at[col_v.at[b]], rows_v.at[b], gsem[b])

            def wait_gather(b):
                pltpu.make_async_copy(x_hbm.at[col_v.at[b]], rows_v.at[b],
                                      gsem[b]).wait()

            def start_scatter(sb):
                pltpu.async_copy(scaled_v.at[sb], acc.at[roff_v.at[sb]], ssem,
                                 add=True)

            def wait_scatter():
                pltpu.make_async_copy(scaled_v.at[0], acc.at[roff_v.at[0]],
                                      ssem).wait()

            def compute(ci, b, sb):
                cb = chunk_base(ci)

                @plsc.parallel_loop(0, K_CHUNK // LANES, 1, unroll=2)
                def group(g):
                    gb = pl.multiple_of(g * LANES, LANES)
                    rvec = row_v[b, pl.ds(gb, LANES)]
                    vvec = val_v[b, pl.ds(gb, LANES)]
                    eglob = cb + gb + lane_iota
                    mask = (eglob >= lo) & (eglob < hi)
                    vvec = jnp.where(mask, vvec, 0.0)
                    roff_v[sb, pl.ds(gb, LANES)] = sbase + jnp.clip(
                        rvec - base_row, 0, rpw - 1)
                    for l in range(LANES):
                        vv = jnp.full((LANES,), vvec[l], jnp.float32)
                        e = gb + l
                        for j in range(d // LANES):
                            js = pl.ds(j * LANES, LANES)
                            scaled_v[sb, e, js] = rows_v[b, e, js] * vv

            @pl.when(nch > 0)
            def _prologue():
                start_idx(0, 0)

                @pl.when(nch > 1)
                def _():
                    start_idx(1, 1)

                @pl.when(nch > 2)
                def _():
                    start_idx(2, 2)

                wait_idx(0)
                start_gather(0)

                @pl.when(nch > 1)
                def _():
                    wait_idx(1)
                    start_gather(1)

            def quad(pp, carry):
                for b in range(NBUF):
                    ci = pp * NBUF + b
                    sb = b % 2

                    @pl.when(ci < nch)
                    def _(ci=ci, b=b, sb=sb):
                        @pl.when(ci + 2 < nch)
                        def _(b2=(b + 2) % NBUF):
                            wait_idx(b2)
                            start_gather(b2)

                        wait_gather(b)
                        compute(ci, b, sb)

                        @pl.when(ci >= 1)
                        def _():
                            wait_scatter()

                        start_scatter(sb)

                        @pl.when(ci + 3 < nch)
                        def _(ci=ci, b3=(b + 3) % NBUF):
                            start_idx(ci + 3, b3)

                return carry

            lax.fori_loop(0, (nch + NBUF - 1) // NBUF, quad, 0)

            @pl.when(nch > 0)
            def _drain():
                wait_scatter()

            if mode == "plain":
                pltpu.sync_copy(acc.at[pl.ds(sbase, rpw)],
                                out_hbm.at[pl.ds(base_row, rpw)])
            elif mode == "combine":
                (a_hbm,) = extras

                def ep(t, carry):
                    r0 = t * ep_rows
                    pltpu.sync_copy(a_hbm.at[pl.ds(base_row + r0, ep_rows)],
                                    ep_v.at[0])
                    pltpu.sync_copy(acc.at[pl.ds(sbase + r0, ep_rows)],
                                    ep_v.at[3])

                    def vbody(rr, c2):
                        for j in range(d // LANES):
                            js = pl.ds(j * LANES, LANES)
                            ep_v[3, rr, js] = (1.1 * ep_v[0, rr, js]
                                               - 0.1 * ep_v[3, rr, js])
                        return c2

                    lax.fori_loop(0, ep_rows, vbody, 0)
                    pltpu.sync_copy(ep_v.at[3],
                                    out_hbm.at[pl.ds(base_row + r0, ep_rows)])
                    return carry

                lax.fori_loop(0, EP_CHUNKS, ep, 0)
            else:
                e0_hbm, e1_hbm, e2_hbm = extras

                def ep(t, carry):
                    r0 = t * ep_rows
                    gr = base_row + r0
                    pltpu.sync_copy(e0_hbm.at[pl.ds(gr, ep_rows)], ep_v.at[0])
                    pltpu.sync_copy(e1_hbm.at[pl.ds(gr, ep_rows)], ep_v.at[1])
                    pltpu.sync_copy(e2_hbm.at[pl.ds(gr, ep_rows)], ep_v.at[2])
                    pltpu.sync_copy(acc.at[pl.ds(sbase + r0, ep_rows)],
                                    ep_v.at[3])

                    def vbody(rr, c2):
                        for j in range(d // LANES):
                            js = pl.ds(j * LANES, LANES)
                            s = (ep_v[0, rr, js] + ep_v[1, rr, js]
                                 + ep_v[2, rr, js] + ep_v[3, rr, js])
                            ep_v[3, rr, js] = 0.25 * s
                        return c2

                    lax.fori_loop(0, ep_rows, vbody, 0)
                    pltpu.sync_copy(ep_v.at[3],
                                    out_hbm.at[pl.ds(gr, ep_rows)])
                    return carry

                lax.fori_loop(0, EP_CHUNKS, ep, 0)

            return pcarry

        lax.fori_loop(0, NPASS, pass_body, 0)

    return spmm


def kernel(user_emb, item_emb, adj_row, adj_col, adj_val,
           ai_row, ai_col, ai_val, aj_row, aj_col, aj_val):
    n = user_emb.shape[0] + item_emb.shape[0]
    user_num = user_emb.shape[0]
    d = user_emb.shape[1]
    nvw = NW * NPASS
    rpw = -(-n // nvw)
    rpw = -(-rpw // (2 * EP_CHUNKS)) * (2 * EP_CHUNKS)
    n_pad = nvw * rpw

    def prep(row, col, val):
        row = row.astype(jnp.int32)
        col = col.astype(jnp.int32)
        val = val.astype(jnp.float32)
        cuts = (jnp.arange(nvw + 1, dtype=jnp.int32) * rpw).astype(row.dtype)
        b = jnp.searchsorted(row, cuts).astype(jnp.int32)
        bounds = jnp.zeros((nvw, LANES), jnp.int32)
        bounds = bounds.at[:, 0].set(b[:-1]).at[:, 1].set(b[1:]).reshape(-1)
        pad = NBUF * K_CHUNK + 8
        row = jnp.pad(row, (0, pad))
        col = jnp.pad(col, (0, pad))
        val = jnp.pad(val, (0, pad))
        return bounds, row, col, val

    adj = prep(adj_row, adj_col, adj_val)
    aj = prep(aj_row, aj_col, aj_val)
    ai = prep(ai_row, ai_col, ai_val)

    x0 = jnp.concatenate([user_emb, item_emb], axis=0)
    x0 = jnp.pad(x0, ((0, n_pad - n), (0, 0)))

    plain = _make_spmm(n_pad, rpw, d, "plain")
    combine = _make_spmm(n_pad, rpw, d, "combine")
    final = _make_spmm(n_pad, rpw, d, "final")

    a1 = plain(*adj, x0)
    b1 = plain(*aj, a1)
    e1 = combine(*ai, b1, a1)
    a2 = plain(*adj, e1)
    b2 = plain(*aj, a2)
    e2 = combine(*ai, b2, a2)
    out = final(*adj, e2, x0, e1, e2)

    return (out[:user_num], out[user_num:n])
